# baseline (speedup 1.0000x reference)
import functools

import jax
import jax.numpy as jnp
from jax.experimental import pallas as pl
from jax.experimental.pallas import tpu as pltpu

_UNROLL = 2
_BLK = 512


def _router_body(*refs, inv_n, unroll, blk):
    x_refs = refs[:unroll]
    w_ref = refs[unroll]
    probs_ref, logits_ref, z_ref = refs[unroll + 1:]

    w = w_ref[...]
    part = 0.0
    for u in range(unroll):
        xb = x_refs[u][...]
        logits = jnp.dot(xb, w, preferred_element_type=jnp.float32)

        m = jnp.max(logits, axis=-1, keepdims=True)
        e = jnp.exp(logits - m)
        rows = pl.ds(u * blk, blk)
        probs_ref[rows, :] = e / jnp.sum(e, axis=-1, keepdims=True)

        absum = jnp.sum(jnp.abs(xb), axis=-1, keepdims=True)
        masked = jnp.where(absum > 0, logits, 0.0)
        logits_ref[rows, :] = masked

        mm = jnp.max(masked, axis=-1, keepdims=True)
        lse = (jnp.log(jnp.sum(jnp.exp(masked - mm), axis=-1,
                               keepdims=True)) + mm)
        part += jnp.sum(lse * lse) * inv_n

    z_ref[...] = jnp.full_like(z_ref, part)


def kernel(x, W):
    b, s, d = x.shape
    n = b * s
    e = W.shape[1]
    xf = x.reshape(n, d)

    blk, unroll = _BLK, _UNROLL
    steps = n // (blk * unroll)

    def x_spec(u):
        return pl.BlockSpec((blk, d), lambda i, u=u: (i * unroll + u, 0))

    body = functools.partial(_router_body, inv_n=1.0 / n, unroll=unroll,
                             blk=blk)
    probs, logits, z = pl.pallas_call(
        body,
        grid=(steps,),
        in_specs=[x_spec(u) for u in range(unroll)]
        + [pl.BlockSpec((d, e), lambda i: (0, 0))],
        out_specs=[
            pl.BlockSpec((unroll * blk, e), lambda i: (i, 0)),
            pl.BlockSpec((unroll * blk, e), lambda i: (i, 0)),
            pl.BlockSpec((1, 1, 1), lambda i: (i, 0, 0)),
        ],
        out_shape=[
            jax.ShapeDtypeStruct((n, e), jnp.float32),
            jax.ShapeDtypeStruct((n, e), jnp.float32),
            jax.ShapeDtypeStruct((steps, 1, 1), jnp.float32),
        ],
        compiler_params=pltpu.CompilerParams(
            dimension_semantics=("parallel",),
        ),
    )(*([xf] * unroll), W)
    return probs, logits, jnp.sum(z)


# speedup vs baseline: 1.0322x; 1.0322x over previous
import functools

import jax
import jax.numpy as jnp
from jax.experimental import pallas as pl

_BLK = 1024


def _router_body(x_ref, w_ref, probs_ref, logits_ref, z_ref, *, inv_n):
    i = pl.program_id(0)
    xb = x_ref[...]
    logits = jnp.dot(xb, w_ref[...],
                     preferred_element_type=jnp.float32)

    m = jnp.max(logits, axis=-1, keepdims=True)
    e = jnp.exp(logits - m)
    probs_ref[...] = e / jnp.sum(e, axis=-1, keepdims=True)

    absum = jnp.sum(jnp.abs(xb), axis=-1, keepdims=True)
    masked = jnp.where(absum > 0, logits, 0.0)
    logits_ref[...] = masked

    mm = jnp.max(masked, axis=-1, keepdims=True)
    lse = jnp.log(jnp.sum(jnp.exp(masked - mm), axis=-1, keepdims=True)) + mm
    part = jnp.sum(lse * lse) * inv_n

    @pl.when(i == 0)
    def _():
        z_ref[...] = jnp.zeros_like(z_ref)

    z_ref[...] = z_ref[...] + part


def kernel(x, W):
    b, s, d = x.shape
    n = b * s
    e = W.shape[1]
    xf = x.reshape(n, d)

    blk = _BLK
    body = functools.partial(_router_body, inv_n=1.0 / n)
    probs, logits, z = pl.pallas_call(
        body,
        grid=(n // blk,),
        in_specs=[
            pl.BlockSpec((blk, d), lambda i: (i, 0)),
            pl.BlockSpec((d, e), lambda i: (0, 0)),
        ],
        out_specs=[
            pl.BlockSpec((blk, e), lambda i: (i, 0)),
            pl.BlockSpec((blk, e), lambda i: (i, 0)),
            pl.BlockSpec((1, 1), lambda i: (0, 0)),
        ],
        out_shape=[
            jax.ShapeDtypeStruct((n, e), jnp.float32),
            jax.ShapeDtypeStruct((n, e), jnp.float32),
            jax.ShapeDtypeStruct((1, 1), jnp.float32),
        ],
    )(xf, W)
    return probs, logits, z[0, 0]


# speedup vs baseline: 1.4048x; 1.3609x over previous
import functools

import jax
import jax.numpy as jnp
from jax.experimental import pallas as pl
from jax.experimental.layout import Format, Layout, with_layout_constraint

_BLK = 1024


def _router_body(x_ref, wt_ref, probs_ref, logits_ref, z_ref, *, inv_n):
    i = pl.program_id(0)
    xb = x_ref[...]
    logits = jax.lax.dot_general(
        xb, wt_ref[...], (((1,), (1,)), ((), ())),
        preferred_element_type=jnp.float32)

    m = jnp.max(logits, axis=-1, keepdims=True)
    e = jnp.exp(logits - m)
    probs_ref[...] = e / jnp.sum(e, axis=-1, keepdims=True)

    absum = jnp.sum(jnp.abs(xb), axis=-1, keepdims=True)
    masked = jnp.where(absum > 0, logits, 0.0)
    logits_ref[...] = masked

    mm = jnp.max(masked, axis=-1, keepdims=True)
    lse = jnp.log(jnp.sum(jnp.exp(masked - mm), axis=-1, keepdims=True)) + mm
    part = jnp.sum(lse * lse) * inv_n

    @pl.when(i == 0)
    def _():
        z_ref[...] = jnp.zeros_like(z_ref)

    z_ref[...] = z_ref[...] + part


def kernel(x, W):
    b, s, d = x.shape
    n = b * s
    e = W.shape[1]
    xf = x.reshape(n, d)

    blk = _BLK
    body = functools.partial(_router_body, inv_n=1.0 / n)
    probs, logits, z = pl.pallas_call(
        body,
        grid=(n // blk,),
        in_specs=[
            pl.BlockSpec((blk, d), lambda i: (i, 0)),
            pl.BlockSpec((e, d), lambda i: (0, 0)),
        ],
        out_specs=[
            pl.BlockSpec((blk, e), lambda i: (i, 0)),
            pl.BlockSpec((blk, e), lambda i: (i, 0)),
            pl.BlockSpec((1, 1), lambda i: (0, 0)),
        ],
        out_shape=[
            jax.ShapeDtypeStruct((n, e), jnp.float32),
            jax.ShapeDtypeStruct((n, e), jnp.float32),
            jax.ShapeDtypeStruct((1, 1), jnp.float32),
        ],
    )(xf, W.T)
    fmt = Layout(major_to_minor=(0, 1))
    probs = with_layout_constraint(probs, fmt)
    logits = with_layout_constraint(logits, fmt)
    return probs, logits, z[0, 0]
